# Initial kernel scaffold; baseline (speedup 1.0000x reference)
import jax
import jax.numpy as jnp
from jax.experimental import pallas as pl


def kernel(x, edge_index, edge_weights, W1, b1, W2, b2, W3, b3):
    raise NotImplementedError("write your pallas kernel here")


# speedup vs baseline: 11.8432x; 11.8432x over previous
import dataclasses
import functools

import jax
import jax.numpy as jnp
from jax import lax
from jax.experimental import pallas as pl
from jax.experimental.pallas import tpu as pltpu
from jax.experimental.pallas import tpu_sc as plsc

NC = 2
NS = 16
NW = NC * NS
LANES = 16

K = 80
ZR = 128
NPAD = 10240


def _sc_params():
    cp = pltpu.CompilerParams()
    if "needs_layout_passes" in pltpu.CompilerParams.__dataclass_fields__:
        cp = dataclasses.replace(cp, needs_layout_passes=False)
    return cp


def _softplus_tc(ew, E):
    def body(e_ref, o_ref):
        v = e_ref[...]
        o_ref[...] = jnp.maximum(v, 0.0) + jnp.log1p(jnp.exp(-jnp.abs(v)))

    ew2 = ew.reshape(E // 128, 128)
    out = pl.pallas_call(
        body,
        out_shape=jax.ShapeDtypeStruct((E // 128, 128), jnp.float32),
    )(ew2)
    return out


def _deg_sc(dst_flat, w_flat, N, E):
    EPW = E // NW
    mesh = plsc.VectorSubcoreMesh(core_axis_name="c", subcore_axis_name="s")

    @functools.partial(
        pl.kernel,
        out_type=jax.ShapeDtypeStruct((NW, 1, N), jnp.float32),
        mesh=mesh,
        compiler_params=_sc_params(),
        scratch_types=[
            pltpu.VMEM((N,), jnp.float32),
            pltpu.VMEM((EPW,), jnp.int32),
            pltpu.VMEM((EPW,), jnp.float32),
        ],
    )
    def k(dst_hbm, w_hbm, dp_hbm, dacc, didx, wbuf):
        c = lax.axis_index("c")
        s = lax.axis_index("s")
        wid = c * NS + s

        @pl.loop(0, N // LANES)
        def _(i):
            dacc[pl.ds(i * LANES, LANES)] = jnp.zeros((LANES,), jnp.float32)

        base = wid * EPW
        pltpu.sync_copy(dst_hbm.at[pl.ds(base, EPW)], didx)
        pltpu.sync_copy(w_hbm.at[pl.ds(base, EPW)], wbuf)

        @pl.loop(0, EPW // LANES)
        def _(t):
            idxv = didx[pl.ds(t * LANES, LANES)]
            wv = wbuf[pl.ds(t * LANES, LANES)]
            plsc.addupdate_scatter(dacc, [idxv], wv)

        pltpu.sync_copy(dacc, dp_hbm.at[wid, 0])

    return k(dst_flat, w_flat).reshape(NW, N)


def _dinv_tc(dp, N):
    def body(dp_ref, dinv_ref):
        deg = jnp.sum(dp_ref[...], axis=0) + 1.0
        dinv = jnp.where(deg > 0, lax.rsqrt(jnp.maximum(deg, 1e-12)), 0.0)
        dinv_ref[...] = dinv[:, None]

    return pl.pallas_call(
        body,
        out_shape=jax.ShapeDtypeStruct((N, 1), jnp.float32),
    )(dp)


def _prep_tc(dinv, x, W1, N, D):
    B = 2000

    def body(dinv_ref, x_ref, w_ref, g_ref):
        z = jnp.dot(x_ref[...], w_ref[...], preferred_element_type=jnp.float32)
        g_ref[...] = z * dinv_ref[...]

    return pl.pallas_call(
        body,
        grid=(N // B,),
        in_specs=[
            pl.BlockSpec((B, 1), lambda i: (i, 0)),
            pl.BlockSpec((B, D), lambda i: (i, 0)),
            pl.BlockSpec((D, D), lambda i: (0, 0)),
        ],
        out_specs=pl.BlockSpec((B, D), lambda i: (i, 0)),
        out_shape=jax.ShapeDtypeStruct((N, D), jnp.float32),
    )(dinv, x, W1)


def _spmm_sc(g, src4, dst4, w4, N, D):
    NB = src4.shape[1]
    CB = src4.shape[2]
    RPW = NPAD // NS
    mesh = plsc.VectorSubcoreMesh(core_axis_name="c", subcore_axis_name="s")

    @functools.partial(
        pl.kernel,
        out_type=jax.ShapeDtypeStruct((NC, NPAD, D), jnp.float32),
        mesh=mesh,
        compiler_params=_sc_params(),
        scratch_types=[
            pltpu.VMEM_SHARED((NPAD, D), jnp.float32),
            pltpu.VMEM((CB, K), jnp.int32),
            pltpu.VMEM((CB, K), jnp.int32),
            pltpu.VMEM((CB, K), jnp.float32),
            pltpu.VMEM((K, D), jnp.float32),
        ],
    )
    def k(g_hbm, src_hbm, dst_hbm, w_hbm, p_hbm, acc, sidx, didx, wbuf, rows):
        c = lax.axis_index("c")
        s = lax.axis_index("s")
        wid = c * NS + s

        @pl.loop(0, K)
        def _(r):
            for q in range(D // LANES):
                rows[r, pl.ds(q * LANES, LANES)] = jnp.zeros((LANES,), jnp.float32)

        @pl.loop(0, RPW // K)
        def _(i):
            pltpu.sync_copy(rows, acc.at[pl.ds(s * RPW + i * K, K)])

        plsc.subcore_barrier()

        @pl.loop(0, NB)
        def _(b):
            pltpu.sync_copy(src_hbm.at[wid, b], sidx)
            pltpu.sync_copy(dst_hbm.at[wid, b], didx)
            pltpu.sync_copy(w_hbm.at[wid, b], wbuf)

            @pl.loop(0, CB)
            def _(ci):
                pltpu.sync_copy(g_hbm.at[sidx.at[ci]], rows)

                @pl.loop(0, K)
                def _(j):
                    wv = plsc.load_gather(
                        wbuf.at[ci], [jnp.full((LANES,), j, jnp.int32)])
                    for q in range(D // LANES):
                        sl = pl.ds(q * LANES, LANES)
                        rows[j, sl] = rows[j, sl] * wv

                pltpu.sync_copy(rows, acc.at[didx.at[ci]], add=True)

        plsc.subcore_barrier()
        pltpu.sync_copy(acc.at[pl.ds(s * RPW, RPW)],
                        p_hbm.at[c, pl.ds(s * RPW, RPW)])

    return k(g, src4, dst4, w4)


def _mid_tc(p, g, dinv, b, Wn, N, D):
    B = 2000

    def body(p_ref, g_ref, dinv_ref, b_ref, w_ref, o_ref):
        dinv_b = dinv_ref[...]
        t = (p_ref[0] + p_ref[1] + g_ref[...]) * dinv_b + b_ref[...]
        t = jnp.maximum(t, 0.0)
        o_ref[...] = (
            jnp.dot(t, w_ref[...], preferred_element_type=jnp.float32) * dinv_b
        )

    return pl.pallas_call(
        body,
        grid=(N // B,),
        in_specs=[
            pl.BlockSpec((NC, B, D), lambda i: (0, i, 0)),
            pl.BlockSpec((B, D), lambda i: (i, 0)),
            pl.BlockSpec((B, 1), lambda i: (i, 0)),
            pl.BlockSpec((1, D), lambda i: (0, 0)),
            pl.BlockSpec((D, D), lambda i: (0, 0)),
        ],
        out_specs=pl.BlockSpec((B, D), lambda i: (i, 0)),
        out_shape=jax.ShapeDtypeStruct((N, D), jnp.float32),
    )(p, g, dinv, b.reshape(1, D), Wn)


def _final_tc(p, g, dinv, b, N, D):
    B = 2000

    def body(p_ref, g_ref, dinv_ref, b_ref, o_ref):
        o_ref[...] = (p_ref[0] + p_ref[1] + g_ref[...]) * dinv_ref[...] + b_ref[...]

    return pl.pallas_call(
        body,
        grid=(N // B,),
        in_specs=[
            pl.BlockSpec((NC, B, D), lambda i: (0, i, 0)),
            pl.BlockSpec((B, D), lambda i: (i, 0)),
            pl.BlockSpec((B, 1), lambda i: (i, 0)),
            pl.BlockSpec((1, D), lambda i: (0, 0)),
        ],
        out_specs=pl.BlockSpec((B, D), lambda i: (i, 0)),
        out_shape=jax.ShapeDtypeStruct((N, D), jnp.float32),
    )(p, g, dinv, b.reshape(1, D))


def kernel(x, edge_index, edge_weights, W1, b1, W2, b2, W3, b3):
    N, D = x.shape
    E = edge_weights.shape[0]
    EPW = E // NW
    CHUNKS = EPW // K

    src = edge_index[0]
    dst = edge_index[1]

    NB = 5
    CB = CHUNKS // NB
    w = _softplus_tc(edge_weights, E)
    w_flat = w.reshape(E)
    src4 = src.reshape(NW, NB, CB, K)
    dst4 = dst.reshape(NW, NB, CB, K)
    w4 = w_flat.reshape(NW, NB, CB, K)

    dp = _deg_sc(dst, w_flat, N, E)
    dinv = _dinv_tc(dp, N)
    g1 = _prep_tc(dinv, x, W1, N, D)
    p1 = _spmm_sc(g1, src4, dst4, w4, N, D)
    g2 = _mid_tc(p1, g1, dinv, b1, W2, N, D)
    p2 = _spmm_sc(g2, src4, dst4, w4, N, D)
    g3 = _mid_tc(p2, g2, dinv, b2, W3, N, D)
    p3 = _spmm_sc(g3, src4, dst4, w4, N, D)
    out = _final_tc(p3, g3, dinv, b3, N, D)
    return out


# speedup vs baseline: 18.9118x; 1.5968x over previous
import dataclasses
import functools

import jax
import jax.numpy as jnp
from jax import lax
from jax.experimental import pallas as pl
from jax.experimental.pallas import tpu as pltpu
from jax.experimental.pallas import tpu_sc as plsc

NC = 2
NS = 16
NW = NC * NS
LANES = 16

K = 125
NPAD = 10240


def _sc_params():
    cp = pltpu.CompilerParams()
    if "needs_layout_passes" in pltpu.CompilerParams.__dataclass_fields__:
        cp = dataclasses.replace(cp, needs_layout_passes=False)
    return cp


def _softplus_tc(ew, E):
    def body(e_ref, o_ref):
        v = e_ref[...]
        o_ref[...] = jnp.maximum(v, 0.0) + jnp.log1p(jnp.exp(-jnp.abs(v)))

    ew2 = ew.reshape(E // 128, 128)
    out = pl.pallas_call(
        body,
        out_shape=jax.ShapeDtypeStruct((E // 128, 128), jnp.float32),
    )(ew2)
    return out


def _deg_sc(dst_flat, w_flat, N, E):
    EPW = E // NW
    mesh = plsc.VectorSubcoreMesh(core_axis_name="c", subcore_axis_name="s")

    @functools.partial(
        pl.kernel,
        out_type=jax.ShapeDtypeStruct((NW, 1, N), jnp.float32),
        mesh=mesh,
        compiler_params=_sc_params(),
        scratch_types=[
            pltpu.VMEM((N,), jnp.float32),
            pltpu.VMEM((EPW,), jnp.int32),
            pltpu.VMEM((EPW,), jnp.float32),
        ],
    )
    def k(dst_hbm, w_hbm, dp_hbm, dacc, didx, wbuf):
        c = lax.axis_index("c")
        s = lax.axis_index("s")
        wid = c * NS + s

        @pl.loop(0, N // LANES)
        def _(i):
            dacc[pl.ds(i * LANES, LANES)] = jnp.zeros((LANES,), jnp.float32)

        base = wid * EPW
        pltpu.sync_copy(dst_hbm.at[pl.ds(base, EPW)], didx)
        pltpu.sync_copy(w_hbm.at[pl.ds(base, EPW)], wbuf)

        @pl.loop(0, EPW // LANES)
        def _(t):
            idxv = didx[pl.ds(t * LANES, LANES)]
            wv = wbuf[pl.ds(t * LANES, LANES)]
            plsc.addupdate_scatter(dacc, [idxv], wv)

        pltpu.sync_copy(dacc, dp_hbm.at[wid, 0])

    return k(dst_flat, w_flat).reshape(NW, N)


def _dinv_tc(dp, N):
    def body(dp_ref, dinv_ref):
        deg = jnp.sum(dp_ref[...], axis=0) + 1.0
        dinv = jnp.where(deg > 0, lax.rsqrt(jnp.maximum(deg, 1e-12)), 0.0)
        dinv_ref[...] = dinv[:, None]

    return pl.pallas_call(
        body,
        out_shape=jax.ShapeDtypeStruct((N, 1), jnp.float32),
    )(dp)


def _prep_tc(dinv, x, W1, N, D):
    B = 2000

    def body(dinv_ref, x_ref, w_ref, g_ref):
        z = jnp.dot(x_ref[...], w_ref[...], preferred_element_type=jnp.float32)
        g_ref[...] = z * dinv_ref[...]

    return pl.pallas_call(
        body,
        grid=(N // B,),
        in_specs=[
            pl.BlockSpec((B, 1), lambda i: (i, 0)),
            pl.BlockSpec((B, D), lambda i: (i, 0)),
            pl.BlockSpec((D, D), lambda i: (0, 0)),
        ],
        out_specs=pl.BlockSpec((B, D), lambda i: (i, 0)),
        out_shape=jax.ShapeDtypeStruct((N, D), jnp.float32),
    )(dinv, x, W1)


def _spmm_sc(g, src4, dst4, w4, N, D):
    NB = src4.shape[1]
    CB = src4.shape[2]
    RPW = NPAD // NS
    mesh = plsc.VectorSubcoreMesh(core_axis_name="c", subcore_axis_name="s")

    @functools.partial(
        pl.kernel,
        out_type=jax.ShapeDtypeStruct((NC, NPAD, D), jnp.float32),
        mesh=mesh,
        compiler_params=_sc_params(),
        scratch_types=[
            pltpu.VMEM_SHARED((NPAD, D), jnp.float32),
            pltpu.VMEM((CB, K), jnp.int32),
            pltpu.VMEM((CB, K), jnp.int32),
            pltpu.VMEM((CB, K), jnp.float32),
            pltpu.VMEM((K, D), jnp.float32),
            pltpu.VMEM((K, D), jnp.float32),
            pltpu.SemaphoreType.DMA,
            pltpu.SemaphoreType.DMA,
            pltpu.SemaphoreType.DMA,
            pltpu.SemaphoreType.DMA,
        ],
    )
    def k(g_hbm, src_hbm, dst_hbm, w_hbm, p_hbm, acc,
          sidx, didx, wbuf, rows0, rows1, gsem0, gsem1, ssem0, ssem1):
        c = lax.axis_index("c")
        s = lax.axis_index("s")
        wid = c * NS + s
        rbufs = (rows0, rows1)
        gsems = (gsem0, gsem1)
        ssems = (ssem0, ssem1)

        @pl.loop(0, 80)
        def _(r):
            for q in range(D // LANES):
                rows0[r, pl.ds(q * LANES, LANES)] = jnp.zeros((LANES,), jnp.float32)

        @pl.loop(0, RPW // 80)
        def _(i):
            pltpu.sync_copy(rows0.at[pl.ds(0, 80)],
                            acc.at[pl.ds(s * RPW + i * 80, 80)])

        plsc.subcore_barrier()

        def scale(rbuf, ci):
            @pl.loop(0, K)
            def _(j):
                wv = plsc.load_gather(
                    wbuf.at[ci], [jnp.full((LANES,), j, jnp.int32)])
                for q in range(D // LANES):
                    sl = pl.ds(q * LANES, LANES)
                    rbuf[j, sl] = rbuf[j, sl] * wv

        @pl.loop(0, NB)
        def _(b):
            pltpu.sync_copy(src_hbm.at[wid, b], sidx)
            pltpu.sync_copy(dst_hbm.at[wid, b], didx)
            pltpu.sync_copy(w_hbm.at[wid, b], wbuf)

            descs = {}
            descs[("g", 0)] = pltpu.async_copy(
                g_hbm.at[sidx.at[0]], rows0, gsem0)
            for ci in range(CB):
                p = ci % 2
                if ci + 1 < CB:
                    if ci >= 1:
                        descs[("s", ci - 1)].wait()
                    descs[("g", ci + 1)] = pltpu.async_copy(
                        g_hbm.at[sidx.at[ci + 1]], rbufs[1 - p], gsems[1 - p])
                descs[("g", ci)].wait()
                scale(rbufs[p], ci)
                descs[("s", ci)] = pltpu.async_copy(
                    rbufs[p], acc.at[didx.at[ci]], ssems[p], add=True)
            descs[("s", CB - 2)].wait()
            descs[("s", CB - 1)].wait()

        plsc.subcore_barrier()
        pltpu.sync_copy(acc.at[pl.ds(s * RPW, RPW)],
                        p_hbm.at[c, pl.ds(s * RPW, RPW)])

    return k(g, src4, dst4, w4)


def _mid_tc(p, g, dinv, b, Wn, N, D):
    B = 2000

    def body(p_ref, g_ref, dinv_ref, b_ref, w_ref, o_ref):
        dinv_b = dinv_ref[...]
        t = (p_ref[0] + p_ref[1] + g_ref[...]) * dinv_b + b_ref[...]
        t = jnp.maximum(t, 0.0)
        o_ref[...] = (
            jnp.dot(t, w_ref[...], preferred_element_type=jnp.float32) * dinv_b
        )

    return pl.pallas_call(
        body,
        grid=(N // B,),
        in_specs=[
            pl.BlockSpec((NC, B, D), lambda i: (0, i, 0)),
            pl.BlockSpec((B, D), lambda i: (i, 0)),
            pl.BlockSpec((B, 1), lambda i: (i, 0)),
            pl.BlockSpec((1, D), lambda i: (0, 0)),
            pl.BlockSpec((D, D), lambda i: (0, 0)),
        ],
        out_specs=pl.BlockSpec((B, D), lambda i: (i, 0)),
        out_shape=jax.ShapeDtypeStruct((N, D), jnp.float32),
    )(p, g, dinv, b.reshape(1, D), Wn)


def _final_tc(p, g, dinv, b, N, D):
    B = 2000

    def body(p_ref, g_ref, dinv_ref, b_ref, o_ref):
        o_ref[...] = (p_ref[0] + p_ref[1] + g_ref[...]) * dinv_ref[...] + b_ref[...]

    return pl.pallas_call(
        body,
        grid=(N // B,),
        in_specs=[
            pl.BlockSpec((NC, B, D), lambda i: (0, i, 0)),
            pl.BlockSpec((B, D), lambda i: (i, 0)),
            pl.BlockSpec((B, 1), lambda i: (i, 0)),
            pl.BlockSpec((1, D), lambda i: (0, 0)),
        ],
        out_specs=pl.BlockSpec((B, D), lambda i: (i, 0)),
        out_shape=jax.ShapeDtypeStruct((N, D), jnp.float32),
    )(p, g, dinv, b.reshape(1, D))


def kernel(x, edge_index, edge_weights, W1, b1, W2, b2, W3, b3):
    N, D = x.shape
    E = edge_weights.shape[0]
    EPW = E // NW
    CHUNKS = EPW // K

    src = edge_index[0]
    dst = edge_index[1]

    NB = 5
    CB = CHUNKS // NB
    w = _softplus_tc(edge_weights, E)
    w_flat = w.reshape(E)
    src4 = src.reshape(NW, NB, CB, K)
    dst4 = dst.reshape(NW, NB, CB, K)
    w4 = w_flat.reshape(NW, NB, CB, K)

    dp = _deg_sc(dst, w_flat, N, E)
    dinv = _dinv_tc(dp, N)
    g1 = _prep_tc(dinv, x, W1, N, D)
    p1 = _spmm_sc(g1, src4, dst4, w4, N, D)
    g2 = _mid_tc(p1, g1, dinv, b1, W2, N, D)
    p2 = _spmm_sc(g2, src4, dst4, w4, N, D)
    g3 = _mid_tc(p2, g2, dinv, b2, W3, N, D)
    p3 = _spmm_sc(g3, src4, dst4, w4, N, D)
    out = _final_tc(p3, g3, dinv, b3, N, D)
    return out


# speedup vs baseline: 26.6526x; 1.4093x over previous
import dataclasses
import functools

import jax
import jax.numpy as jnp
from jax import lax
from jax.experimental import pallas as pl
from jax.experimental.pallas import tpu as pltpu
from jax.experimental.pallas import tpu_sc as plsc

NC = 2
NS = 16
NW = NC * NS
LANES = 16

K = 125
NPAD = 10240


def _sc_params():
    cp = pltpu.CompilerParams()
    if "needs_layout_passes" in pltpu.CompilerParams.__dataclass_fields__:
        cp = dataclasses.replace(cp, needs_layout_passes=False)
    return cp


def _softplus_tc(ew, E):
    def body(e_ref, o_ref, flag_ref, sqc_ref):
        v = e_ref[...]
        w = jnp.maximum(v, 0.0) + jnp.log1p(jnp.exp(-jnp.abs(v)))
        o_ref[...] = w
        uniform = jnp.min(w) == jnp.max(w)
        flag_ref[...] = jnp.where(uniform, 0, 1) * jnp.ones((1, 128), jnp.int32)
        sqc_ref[...] = jnp.where(uniform, jnp.sqrt(jnp.maximum(jnp.max(w), 0.0)),
                                 1.0) * jnp.ones((1, 1), jnp.float32)

    w, flag, sqc = pl.pallas_call(
        body,
        out_shape=[
            jax.ShapeDtypeStruct((E // 128, 128), jnp.float32),
            jax.ShapeDtypeStruct((1, 128), jnp.int32),
            jax.ShapeDtypeStruct((1, 1), jnp.float32),
        ],
    )(ew.reshape(E // 128, 128))
    return w, flag, sqc


def _deg_sc(dst_flat, w_flat, N, E):
    EPW = E // NW
    mesh = plsc.VectorSubcoreMesh(core_axis_name="c", subcore_axis_name="s")

    @functools.partial(
        pl.kernel,
        out_type=jax.ShapeDtypeStruct((NW, 1, N), jnp.float32),
        mesh=mesh,
        compiler_params=_sc_params(),
        scratch_types=[
            pltpu.VMEM((N,), jnp.float32),
            pltpu.VMEM((EPW,), jnp.int32),
            pltpu.VMEM((EPW,), jnp.float32),
        ],
    )
    def k(dst_hbm, w_hbm, dp_hbm, dacc, didx, wbuf):
        c = lax.axis_index("c")
        s = lax.axis_index("s")
        wid = c * NS + s

        @pl.loop(0, N // LANES)
        def _(i):
            dacc[pl.ds(i * LANES, LANES)] = jnp.zeros((LANES,), jnp.float32)

        base = wid * EPW
        pltpu.sync_copy(dst_hbm.at[pl.ds(base, EPW)], didx)
        pltpu.sync_copy(w_hbm.at[pl.ds(base, EPW)], wbuf)

        @pl.loop(0, EPW // LANES)
        def _(t):
            idxv = didx[pl.ds(t * LANES, LANES)]
            wv = wbuf[pl.ds(t * LANES, LANES)]
            plsc.addupdate_scatter(dacc, [idxv], wv)

        pltpu.sync_copy(dacc, dp_hbm.at[wid, 0])

    return k(dst_flat, w_flat).reshape(NW, N)


def _dinv_tc(dp, N):
    def body(dp_ref, dinv_ref):
        deg = jnp.sum(dp_ref[...], axis=0) + 1.0
        dinv = jnp.where(deg > 0, lax.rsqrt(jnp.maximum(deg, 1e-12)), 0.0)
        dinv_ref[...] = dinv[:, None]

    return pl.pallas_call(
        body,
        out_shape=jax.ShapeDtypeStruct((N, 1), jnp.float32),
    )(dp)


def _prep_tc(dinv, x, W1, sqc, N, D):
    B = 2000

    def body(dinv_ref, x_ref, w_ref, sqc_ref, g_ref, gs_ref):
        z = jnp.dot(x_ref[...], w_ref[...], preferred_element_type=jnp.float32)
        g = z * dinv_ref[...]
        g_ref[...] = g
        gs_ref[...] = g * sqc_ref[0, 0]

    return pl.pallas_call(
        body,
        grid=(N // B,),
        in_specs=[
            pl.BlockSpec((B, 1), lambda i: (i, 0)),
            pl.BlockSpec((B, D), lambda i: (i, 0)),
            pl.BlockSpec((D, D), lambda i: (0, 0)),
            pl.BlockSpec((1, 1), lambda i: (0, 0)),
        ],
        out_specs=[
            pl.BlockSpec((B, D), lambda i: (i, 0)),
            pl.BlockSpec((B, D), lambda i: (i, 0)),
        ],
        out_shape=[
            jax.ShapeDtypeStruct((N, D), jnp.float32),
            jax.ShapeDtypeStruct((N, D), jnp.float32),
        ],
    )(dinv, x, W1, sqc)


def _spmm_sc(g, src4, dst4, w4, flag, N, D):
    NB = src4.shape[1]
    CB = src4.shape[2]
    RPW = NPAD // NS
    mesh = plsc.VectorSubcoreMesh(core_axis_name="c", subcore_axis_name="s")

    @functools.partial(
        pl.kernel,
        out_type=jax.ShapeDtypeStruct((NC, NPAD, D), jnp.float32),
        mesh=mesh,
        compiler_params=_sc_params(),
        scratch_types=[
            pltpu.VMEM_SHARED((NPAD, D), jnp.float32),
            pltpu.VMEM((CB, K), jnp.int32),
            pltpu.VMEM((CB, K), jnp.int32),
            pltpu.VMEM((CB, K), jnp.float32),
            pltpu.VMEM((K, D), jnp.float32),
            pltpu.VMEM((K, D), jnp.float32),
            pltpu.VMEM((LANES,), jnp.int32),
            pltpu.SemaphoreType.DMA,
            pltpu.SemaphoreType.DMA,
            pltpu.SemaphoreType.DMA,
            pltpu.SemaphoreType.DMA,
        ],
    )
    def k(g_hbm, src_hbm, dst_hbm, w_hbm, flag_hbm, p_hbm, acc,
          sidx, didx, wbuf, rows0, rows1, fbuf, gsem0, gsem1, ssem0, ssem1):
        c = lax.axis_index("c")
        s = lax.axis_index("s")
        wid = c * NS + s
        rbufs = (rows0, rows1)
        gsems = (gsem0, gsem1)
        ssems = (ssem0, ssem1)

        @pl.loop(0, 80)
        def _(r):
            for q in range(D // LANES):
                rows0[r, pl.ds(q * LANES, LANES)] = jnp.zeros((LANES,), jnp.float32)

        @pl.loop(0, RPW // 80)
        def _(i):
            pltpu.sync_copy(rows0.at[pl.ds(0, 80)],
                            acc.at[pl.ds(s * RPW + i * 80, 80)])

        pltpu.sync_copy(flag_hbm.at[pl.ds(0, LANES)], fbuf)
        needs_scale = jnp.max(fbuf[...]) > 0
        plsc.subcore_barrier()

        def scale(rbuf, ci):
            @pl.loop(0, K)
            def _(j):
                wv = plsc.load_gather(
                    wbuf.at[ci], [jnp.full((LANES,), j, jnp.int32)])
                for q in range(D // LANES):
                    sl = pl.ds(q * LANES, LANES)
                    rbuf[j, sl] = rbuf[j, sl] * wv

        @pl.loop(0, NB)
        def _(b):
            pltpu.sync_copy(src_hbm.at[wid, b], sidx)
            pltpu.sync_copy(dst_hbm.at[wid, b], didx)
            pltpu.sync_copy(w_hbm.at[wid, b], wbuf)

            descs = {}
            descs[("g", 0)] = pltpu.async_copy(
                g_hbm.at[sidx.at[0]], rows0, gsem0)
            for ci in range(CB):
                p = ci % 2
                if ci + 1 < CB:
                    if ci >= 1:
                        descs[("s", ci - 1)].wait()
                    descs[("g", ci + 1)] = pltpu.async_copy(
                        g_hbm.at[sidx.at[ci + 1]], rbufs[1 - p], gsems[1 - p])
                descs[("g", ci)].wait()

                @pl.when(needs_scale)
                def _(rbuf=rbufs[p], ci=ci):
                    scale(rbuf, ci)

                descs[("s", ci)] = pltpu.async_copy(
                    rbufs[p], acc.at[didx.at[ci]], ssems[p], add=True)
            descs[("s", CB - 2)].wait()
            descs[("s", CB - 1)].wait()

        plsc.subcore_barrier()
        pltpu.sync_copy(acc.at[pl.ds(s * RPW, RPW)],
                        p_hbm.at[c, pl.ds(s * RPW, RPW)])

    return k(g, src4, dst4, w4, flag)


def _mid_tc(p, g, dinv, b, Wn, sqc, N, D):
    B = 2000

    def body(p_ref, g_ref, dinv_ref, b_ref, w_ref, sqc_ref, o_ref, os_ref):
        dinv_b = dinv_ref[...]
        sqc_s = sqc_ref[0, 0]
        t = (sqc_s * (p_ref[0] + p_ref[1]) + g_ref[...]) * dinv_b + b_ref[...]
        t = jnp.maximum(t, 0.0)
        g = jnp.dot(t, w_ref[...], preferred_element_type=jnp.float32) * dinv_b
        o_ref[...] = g
        os_ref[...] = g * sqc_s

    return pl.pallas_call(
        body,
        grid=(N // B,),
        in_specs=[
            pl.BlockSpec((NC, B, D), lambda i: (0, i, 0)),
            pl.BlockSpec((B, D), lambda i: (i, 0)),
            pl.BlockSpec((B, 1), lambda i: (i, 0)),
            pl.BlockSpec((1, D), lambda i: (0, 0)),
            pl.BlockSpec((D, D), lambda i: (0, 0)),
            pl.BlockSpec((1, 1), lambda i: (0, 0)),
        ],
        out_specs=[
            pl.BlockSpec((B, D), lambda i: (i, 0)),
            pl.BlockSpec((B, D), lambda i: (i, 0)),
        ],
        out_shape=[
            jax.ShapeDtypeStruct((N, D), jnp.float32),
            jax.ShapeDtypeStruct((N, D), jnp.float32),
        ],
    )(p, g, dinv, b.reshape(1, D), Wn, sqc)


def _final_tc(p, g, dinv, b, sqc, N, D):
    B = 2000

    def body(p_ref, g_ref, dinv_ref, b_ref, sqc_ref, o_ref):
        o_ref[...] = (
            sqc_ref[0, 0] * (p_ref[0] + p_ref[1]) + g_ref[...]
        ) * dinv_ref[...] + b_ref[...]

    return pl.pallas_call(
        body,
        grid=(N // B,),
        in_specs=[
            pl.BlockSpec((NC, B, D), lambda i: (0, i, 0)),
            pl.BlockSpec((B, D), lambda i: (i, 0)),
            pl.BlockSpec((B, 1), lambda i: (i, 0)),
            pl.BlockSpec((1, D), lambda i: (0, 0)),
            pl.BlockSpec((1, 1), lambda i: (0, 0)),
        ],
        out_specs=pl.BlockSpec((B, D), lambda i: (i, 0)),
        out_shape=jax.ShapeDtypeStruct((N, D), jnp.float32),
    )(p, g, dinv, b.reshape(1, D), sqc)


def kernel(x, edge_index, edge_weights, W1, b1, W2, b2, W3, b3):
    N, D = x.shape
    E = edge_weights.shape[0]
    EPW = E // NW
    CHUNKS = EPW // K

    src = edge_index[0]
    dst = edge_index[1]

    NB = 5
    CB = CHUNKS // NB
    w, flag, sqc = _softplus_tc(edge_weights, E)
    w_flat = w.reshape(E)
    flag_flat = flag.reshape(128)
    src4 = src.reshape(NW, NB, CB, K)
    dst4 = dst.reshape(NW, NB, CB, K)
    w4 = w_flat.reshape(NW, NB, CB, K)

    dp = _deg_sc(dst, w_flat, N, E)
    dinv = _dinv_tc(dp, N)
    g1, gs1 = _prep_tc(dinv, x, W1, sqc, N, D)
    p1 = _spmm_sc(gs1, src4, dst4, w4, flag_flat, N, D)
    g2, gs2 = _mid_tc(p1, g1, dinv, b1, W2, sqc, N, D)
    p2 = _spmm_sc(gs2, src4, dst4, w4, flag_flat, N, D)
    g3, gs3 = _mid_tc(p2, g2, dinv, b2, W3, sqc, N, D)
    p3 = _spmm_sc(gs3, src4, dst4, w4, flag_flat, N, D)
    out = _final_tc(p3, g3, dinv, b3, sqc, N, D)
    return out


# speedup vs baseline: 28.1686x; 1.0569x over previous
import dataclasses
import functools

import jax
import jax.numpy as jnp
from jax import lax
from jax.experimental import pallas as pl
from jax.experimental.pallas import tpu as pltpu
from jax.experimental.pallas import tpu_sc as plsc

NC = 2
NS = 16
NW = NC * NS
LANES = 16

K = 80
NBUF = 3
NPAD = 10240


def _sc_params():
    cp = pltpu.CompilerParams()
    if "needs_layout_passes" in pltpu.CompilerParams.__dataclass_fields__:
        cp = dataclasses.replace(cp, needs_layout_passes=False)
    return cp


def _softplus_tc(ew, E):
    def body(e_ref, o_ref, flag_ref, sqc_ref):
        v = e_ref[...]
        w = jnp.maximum(v, 0.0) + jnp.log1p(jnp.exp(-jnp.abs(v)))
        o_ref[...] = w
        uniform = jnp.min(w) == jnp.max(w)
        flag_ref[...] = jnp.where(uniform, 0, 1) * jnp.ones((1, 128), jnp.int32)
        sqc_ref[...] = jnp.where(uniform, jnp.sqrt(jnp.maximum(jnp.max(w), 0.0)),
                                 1.0) * jnp.ones((1, 1), jnp.float32)

    w, flag, sqc = pl.pallas_call(
        body,
        out_shape=[
            jax.ShapeDtypeStruct((E // 128, 128), jnp.float32),
            jax.ShapeDtypeStruct((1, 128), jnp.int32),
            jax.ShapeDtypeStruct((1, 1), jnp.float32),
        ],
    )(ew.reshape(E // 128, 128))
    return w, flag, sqc


def _deg_sc(dst_flat, w_flat, N, E):
    EPW = E // NW
    mesh = plsc.VectorSubcoreMesh(core_axis_name="c", subcore_axis_name="s")

    @functools.partial(
        pl.kernel,
        out_type=jax.ShapeDtypeStruct((NW, 1, N), jnp.float32),
        mesh=mesh,
        compiler_params=_sc_params(),
        scratch_types=[
            pltpu.VMEM((N,), jnp.float32),
            pltpu.VMEM((EPW,), jnp.int32),
            pltpu.VMEM((EPW,), jnp.float32),
        ],
    )
    def k(dst_hbm, w_hbm, dp_hbm, dacc, didx, wbuf):
        c = lax.axis_index("c")
        s = lax.axis_index("s")
        wid = c * NS + s

        @pl.loop(0, N // LANES)
        def _(i):
            dacc[pl.ds(i * LANES, LANES)] = jnp.zeros((LANES,), jnp.float32)

        base = wid * EPW
        pltpu.sync_copy(dst_hbm.at[pl.ds(base, EPW)], didx)
        pltpu.sync_copy(w_hbm.at[pl.ds(base, EPW)], wbuf)

        @pl.loop(0, EPW // LANES)
        def _(t):
            idxv = didx[pl.ds(t * LANES, LANES)]
            wv = wbuf[pl.ds(t * LANES, LANES)]
            plsc.addupdate_scatter(dacc, [idxv], wv)

        pltpu.sync_copy(dacc, dp_hbm.at[wid, 0])

    return k(dst_flat, w_flat).reshape(NW, N)


def _dinv_tc(dp, N):
    def body(dp_ref, dinv_ref):
        deg = jnp.sum(dp_ref[...], axis=0) + 1.0
        dinv = jnp.where(deg > 0, lax.rsqrt(jnp.maximum(deg, 1e-12)), 0.0)
        dinv_ref[...] = dinv[:, None]

    return pl.pallas_call(
        body,
        out_shape=jax.ShapeDtypeStruct((N, 1), jnp.float32),
    )(dp)


def _prep_tc(dinv, x, W1, sqc, N, D):
    B = 2000

    def body(dinv_ref, x_ref, w_ref, sqc_ref, g_ref, gs_ref):
        z = jnp.dot(x_ref[...], w_ref[...], preferred_element_type=jnp.float32)
        g = z * dinv_ref[...]
        g_ref[...] = g
        gs_ref[...] = g * sqc_ref[0, 0]

    return pl.pallas_call(
        body,
        grid=(N // B,),
        in_specs=[
            pl.BlockSpec((B, 1), lambda i: (i, 0)),
            pl.BlockSpec((B, D), lambda i: (i, 0)),
            pl.BlockSpec((D, D), lambda i: (0, 0)),
            pl.BlockSpec((1, 1), lambda i: (0, 0)),
        ],
        out_specs=[
            pl.BlockSpec((B, D), lambda i: (i, 0)),
            pl.BlockSpec((B, D), lambda i: (i, 0)),
        ],
        out_shape=[
            jax.ShapeDtypeStruct((N, D), jnp.float32),
            jax.ShapeDtypeStruct((N, D), jnp.float32),
        ],
    )(dinv, x, W1, sqc)


def _spmm_sc(g, src4, dst4, w4, flag, N, D):
    NB = src4.shape[1]
    CB = src4.shape[2]
    RPW = NPAD // NS
    mesh = plsc.VectorSubcoreMesh(core_axis_name="c", subcore_axis_name="s")

    @functools.partial(
        pl.kernel,
        out_type=jax.ShapeDtypeStruct((NC, NPAD, D), jnp.float32),
        mesh=mesh,
        compiler_params=_sc_params(),
        scratch_types=[
            pltpu.VMEM_SHARED((NPAD, D), jnp.float32),
            pltpu.VMEM((CB, K), jnp.int32),
            pltpu.VMEM((CB, K), jnp.int32),
            pltpu.VMEM((CB, K), jnp.float32),
            pltpu.VMEM((K, D), jnp.float32),
            pltpu.VMEM((K, D), jnp.float32),
            pltpu.VMEM((K, D), jnp.float32),
            pltpu.VMEM((LANES,), jnp.int32),
            pltpu.SemaphoreType.DMA,
            pltpu.SemaphoreType.DMA,
            pltpu.SemaphoreType.DMA,
            pltpu.SemaphoreType.DMA,
            pltpu.SemaphoreType.DMA,
            pltpu.SemaphoreType.DMA,
        ],
    )
    def k(g_hbm, src_hbm, dst_hbm, w_hbm, flag_hbm, p_hbm, acc,
          sidx, didx, wbuf, rows0, rows1, rows2, fbuf,
          gsem0, gsem1, gsem2, ssem0, ssem1, ssem2):
        c = lax.axis_index("c")
        s = lax.axis_index("s")
        wid = c * NS + s
        rbufs = (rows0, rows1, rows2)
        gsems = (gsem0, gsem1, gsem2)
        ssems = (ssem0, ssem1, ssem2)

        @pl.loop(0, 80)
        def _(r):
            for q in range(D // LANES):
                rows0[r, pl.ds(q * LANES, LANES)] = jnp.zeros((LANES,), jnp.float32)

        @pl.loop(0, RPW // 80)
        def _(i):
            pltpu.sync_copy(rows0.at[pl.ds(0, 80)],
                            acc.at[pl.ds(s * RPW + i * 80, 80)])

        pltpu.sync_copy(flag_hbm.at[pl.ds(0, LANES)], fbuf)
        needs_scale = jnp.max(fbuf[...]) > 0
        plsc.subcore_barrier()

        def scale(rbuf, ci):
            @pl.loop(0, K)
            def _(j):
                wv = plsc.load_gather(
                    wbuf.at[ci], [jnp.full((LANES,), j, jnp.int32)])
                for q in range(D // LANES):
                    sl = pl.ds(q * LANES, LANES)
                    rbuf[j, sl] = rbuf[j, sl] * wv

        @pl.loop(0, NB)
        def _(b):
            pltpu.sync_copy(src_hbm.at[wid, b], sidx)
            pltpu.sync_copy(dst_hbm.at[wid, b], didx)
            pltpu.sync_copy(w_hbm.at[wid, b], wbuf)

            descs = {}
            for ci in range(min(NBUF - 1, CB)):
                descs[("g", ci)] = pltpu.async_copy(
                    g_hbm.at[sidx.at[ci]], rbufs[ci % NBUF], gsems[ci % NBUF])
            for ci in range(CB):
                p = ci % NBUF
                la = ci + NBUF - 1
                if la < CB:
                    if ci >= 1:
                        descs[("s", ci - 1)].wait()
                    descs[("g", la)] = pltpu.async_copy(
                        g_hbm.at[sidx.at[la]], rbufs[la % NBUF], gsems[la % NBUF])
                descs[("g", ci)].wait()

                @pl.when(needs_scale)
                def _(rbuf=rbufs[p], ci=ci):
                    scale(rbuf, ci)

                descs[("s", ci)] = pltpu.async_copy(
                    rbufs[p], acc.at[didx.at[ci]], ssems[p], add=True)
            for ci in range(max(0, CB - NBUF), CB):
                descs[("s", ci)].wait()

        plsc.subcore_barrier()
        pltpu.sync_copy(acc.at[pl.ds(s * RPW, RPW)],
                        p_hbm.at[c, pl.ds(s * RPW, RPW)])

    return k(g, src4, dst4, w4, flag)


def _mid_tc(p, g, dinv, b, Wn, sqc, N, D):
    B = 2000

    def body(p_ref, g_ref, dinv_ref, b_ref, w_ref, sqc_ref, o_ref, os_ref):
        dinv_b = dinv_ref[...]
        sqc_s = sqc_ref[0, 0]
        t = (sqc_s * (p_ref[0] + p_ref[1]) + g_ref[...]) * dinv_b + b_ref[...]
        t = jnp.maximum(t, 0.0)
        g = jnp.dot(t, w_ref[...], preferred_element_type=jnp.float32) * dinv_b
        o_ref[...] = g
        os_ref[...] = g * sqc_s

    return pl.pallas_call(
        body,
        grid=(N // B,),
        in_specs=[
            pl.BlockSpec((NC, B, D), lambda i: (0, i, 0)),
            pl.BlockSpec((B, D), lambda i: (i, 0)),
            pl.BlockSpec((B, 1), lambda i: (i, 0)),
            pl.BlockSpec((1, D), lambda i: (0, 0)),
            pl.BlockSpec((D, D), lambda i: (0, 0)),
            pl.BlockSpec((1, 1), lambda i: (0, 0)),
        ],
        out_specs=[
            pl.BlockSpec((B, D), lambda i: (i, 0)),
            pl.BlockSpec((B, D), lambda i: (i, 0)),
        ],
        out_shape=[
            jax.ShapeDtypeStruct((N, D), jnp.float32),
            jax.ShapeDtypeStruct((N, D), jnp.float32),
        ],
    )(p, g, dinv, b.reshape(1, D), Wn, sqc)


def _final_tc(p, g, dinv, b, sqc, N, D):
    B = 2000

    def body(p_ref, g_ref, dinv_ref, b_ref, sqc_ref, o_ref):
        o_ref[...] = (
            sqc_ref[0, 0] * (p_ref[0] + p_ref[1]) + g_ref[...]
        ) * dinv_ref[...] + b_ref[...]

    return pl.pallas_call(
        body,
        grid=(N // B,),
        in_specs=[
            pl.BlockSpec((NC, B, D), lambda i: (0, i, 0)),
            pl.BlockSpec((B, D), lambda i: (i, 0)),
            pl.BlockSpec((B, 1), lambda i: (i, 0)),
            pl.BlockSpec((1, D), lambda i: (0, 0)),
            pl.BlockSpec((1, 1), lambda i: (0, 0)),
        ],
        out_specs=pl.BlockSpec((B, D), lambda i: (i, 0)),
        out_shape=jax.ShapeDtypeStruct((N, D), jnp.float32),
    )(p, g, dinv, b.reshape(1, D), sqc)


def kernel(x, edge_index, edge_weights, W1, b1, W2, b2, W3, b3):
    N, D = x.shape
    E = edge_weights.shape[0]
    EPW = E // NW
    CHUNKS = EPW // K

    src = edge_index[0]
    dst = edge_index[1]

    NB = 5
    CB = CHUNKS // NB
    w, flag, sqc = _softplus_tc(edge_weights, E)
    w_flat = w.reshape(E)
    flag_flat = flag.reshape(128)
    src4 = src.reshape(NW, NB, CB, K)
    dst4 = dst.reshape(NW, NB, CB, K)
    w4 = w_flat.reshape(NW, NB, CB, K)

    dp = _deg_sc(dst, w_flat, N, E)
    dinv = _dinv_tc(dp, N)
    g1, gs1 = _prep_tc(dinv, x, W1, sqc, N, D)
    p1 = _spmm_sc(gs1, src4, dst4, w4, flag_flat, N, D)
    g2, gs2 = _mid_tc(p1, g1, dinv, b1, W2, sqc, N, D)
    p2 = _spmm_sc(gs2, src4, dst4, w4, flag_flat, N, D)
    g3, gs3 = _mid_tc(p2, g2, dinv, b2, W3, sqc, N, D)
    p3 = _spmm_sc(gs3, src4, dst4, w4, flag_flat, N, D)
    out = _final_tc(p3, g3, dinv, b3, sqc, N, D)
    return out


# speedup vs baseline: 28.4989x; 1.0117x over previous
import dataclasses
import functools

import jax
import jax.numpy as jnp
from jax import lax
from jax.experimental import pallas as pl
from jax.experimental.pallas import tpu as pltpu
from jax.experimental.pallas import tpu_sc as plsc

NC = 2
NS = 16
NW = NC * NS
LANES = 16

K = 80
NBUF = 3
NPAD = 10240


def _sc_params():
    cp = pltpu.CompilerParams()
    if "needs_layout_passes" in pltpu.CompilerParams.__dataclass_fields__:
        cp = dataclasses.replace(cp, needs_layout_passes=False)
    return cp


def _softplus_tc(ew, E):
    def body(e_ref, o_ref, flag_ref, sqc_ref):
        v = e_ref[...]
        w = jnp.maximum(v, 0.0) + jnp.log1p(jnp.exp(-jnp.abs(v)))
        o_ref[...] = w
        uniform = jnp.min(w) == jnp.max(w)
        flag_ref[...] = jnp.where(uniform, 0, 1) * jnp.ones((1, 128), jnp.int32)
        sqc_ref[...] = jnp.where(uniform, jnp.sqrt(jnp.maximum(jnp.max(w), 0.0)),
                                 1.0) * jnp.ones((1, 1), jnp.float32)

    w, flag, sqc = pl.pallas_call(
        body,
        out_shape=[
            jax.ShapeDtypeStruct((E // 128, 128), jnp.float32),
            jax.ShapeDtypeStruct((1, 128), jnp.int32),
            jax.ShapeDtypeStruct((1, 1), jnp.float32),
        ],
    )(ew.reshape(E // 128, 128))
    return w, flag, sqc


def _deg_sc(dst_flat, w_flat, N, E):
    EPW = E // NW
    mesh = plsc.VectorSubcoreMesh(core_axis_name="c", subcore_axis_name="s")

    @functools.partial(
        pl.kernel,
        out_type=jax.ShapeDtypeStruct((NW, 1, N), jnp.float32),
        mesh=mesh,
        compiler_params=_sc_params(),
        scratch_types=[
            pltpu.VMEM((N,), jnp.float32),
            pltpu.VMEM((EPW,), jnp.int32),
            pltpu.VMEM((EPW,), jnp.float32),
        ],
    )
    def k(dst_hbm, w_hbm, dp_hbm, dacc, didx, wbuf):
        c = lax.axis_index("c")
        s = lax.axis_index("s")
        wid = c * NS + s

        @pl.loop(0, N // LANES)
        def _(i):
            dacc[pl.ds(i * LANES, LANES)] = jnp.zeros((LANES,), jnp.float32)

        base = wid * EPW
        pltpu.sync_copy(dst_hbm.at[pl.ds(base, EPW)], didx)
        pltpu.sync_copy(w_hbm.at[pl.ds(base, EPW)], wbuf)

        @pl.loop(0, EPW // LANES)
        def _(t):
            idxv = didx[pl.ds(t * LANES, LANES)]
            wv = wbuf[pl.ds(t * LANES, LANES)]
            plsc.addupdate_scatter(dacc, [idxv], wv)

        pltpu.sync_copy(dacc, dp_hbm.at[wid, 0])

    return k(dst_flat, w_flat).reshape(NW, N)


def _prep_tc(dp, x, W1, sqc, N, D):
    def body(dp_ref, x_ref, w_ref, sqc_ref, g_ref, gs_ref, dinv_ref):
        deg = jnp.sum(dp_ref[...], axis=0) + 1.0
        dinv = jnp.where(deg > 0, lax.rsqrt(jnp.maximum(deg, 1e-12)), 0.0)
        z = jnp.dot(x_ref[...], w_ref[...], preferred_element_type=jnp.float32)
        g = z * dinv[:, None]
        g_ref[...] = g
        gs_ref[...] = g * sqc_ref[0, 0]
        dinv_ref[...] = dinv[:, None]

    return pl.pallas_call(
        body,
        out_shape=[
            jax.ShapeDtypeStruct((N, D), jnp.float32),
            jax.ShapeDtypeStruct((N, D), jnp.float32),
            jax.ShapeDtypeStruct((N, 1), jnp.float32),
        ],
    )(dp, x, W1, sqc)


def _spmm_sc(g, src4, dst4, w4, flag, N, D):
    NB = src4.shape[1]
    CB = src4.shape[2]
    RPW = NPAD // NS
    mesh = plsc.VectorSubcoreMesh(core_axis_name="c", subcore_axis_name="s")

    @functools.partial(
        pl.kernel,
        out_type=jax.ShapeDtypeStruct((NC, NPAD, D), jnp.float32),
        mesh=mesh,
        compiler_params=_sc_params(),
        scratch_types=[
            pltpu.VMEM_SHARED((NPAD, D), jnp.float32),
            pltpu.VMEM((CB, K), jnp.int32),
            pltpu.VMEM((CB, K), jnp.int32),
            pltpu.VMEM((CB, K), jnp.float32),
            pltpu.VMEM((K, D), jnp.float32),
            pltpu.VMEM((K, D), jnp.float32),
            pltpu.VMEM((K, D), jnp.float32),
            pltpu.VMEM((LANES,), jnp.int32),
            pltpu.SemaphoreType.DMA,
            pltpu.SemaphoreType.DMA,
            pltpu.SemaphoreType.DMA,
            pltpu.SemaphoreType.DMA,
            pltpu.SemaphoreType.DMA,
            pltpu.SemaphoreType.DMA,
        ],
    )
    def k(g_hbm, src_hbm, dst_hbm, w_hbm, flag_hbm, p_hbm, acc,
          sidx, didx, wbuf, rows0, rows1, rows2, fbuf,
          gsem0, gsem1, gsem2, ssem0, ssem1, ssem2):
        c = lax.axis_index("c")
        s = lax.axis_index("s")
        wid = c * NS + s
        rbufs = (rows0, rows1, rows2)
        gsems = (gsem0, gsem1, gsem2)
        ssems = (ssem0, ssem1, ssem2)

        @pl.loop(0, 80)
        def _(r):
            for q in range(D // LANES):
                rows0[r, pl.ds(q * LANES, LANES)] = jnp.zeros((LANES,), jnp.float32)

        @pl.loop(0, RPW // 80)
        def _(i):
            pltpu.sync_copy(rows0.at[pl.ds(0, 80)],
                            acc.at[pl.ds(s * RPW + i * 80, 80)])

        pltpu.sync_copy(flag_hbm.at[pl.ds(0, LANES)], fbuf)
        needs_scale = jnp.max(fbuf[...]) > 0
        plsc.subcore_barrier()

        def scale(rbuf, ci):
            @pl.loop(0, K)
            def _(j):
                wv = plsc.load_gather(
                    wbuf.at[ci], [jnp.full((LANES,), j, jnp.int32)])
                for q in range(D // LANES):
                    sl = pl.ds(q * LANES, LANES)
                    rbuf[j, sl] = rbuf[j, sl] * wv

        @pl.loop(0, NB)
        def _(b):
            pltpu.sync_copy(src_hbm.at[wid, b], sidx)
            pltpu.sync_copy(dst_hbm.at[wid, b], didx)
            pltpu.sync_copy(w_hbm.at[wid, b], wbuf)

            descs = {}
            for ci in range(min(NBUF - 1, CB)):
                descs[("g", ci)] = pltpu.async_copy(
                    g_hbm.at[sidx.at[ci]], rbufs[ci % NBUF], gsems[ci % NBUF])
            for ci in range(CB):
                p = ci % NBUF
                la = ci + NBUF - 1
                if la < CB:
                    if ci >= 1:
                        descs[("s", ci - 1)].wait()
                    descs[("g", la)] = pltpu.async_copy(
                        g_hbm.at[sidx.at[la]], rbufs[la % NBUF], gsems[la % NBUF])
                descs[("g", ci)].wait()

                @pl.when(needs_scale)
                def _(rbuf=rbufs[p], ci=ci):
                    scale(rbuf, ci)

                descs[("s", ci)] = pltpu.async_copy(
                    rbufs[p], acc.at[didx.at[ci]], ssems[p], add=True)
            for ci in range(max(0, CB - NBUF), CB):
                descs[("s", ci)].wait()

        plsc.subcore_barrier()
        pltpu.sync_copy(acc.at[pl.ds(s * RPW, RPW)],
                        p_hbm.at[c, pl.ds(s * RPW, RPW)])

    return k(g, src4, dst4, w4, flag)


def _mid_tc(p, g, dinv, b, Wn, sqc, N, D):
    B = 2000

    def body(p_ref, g_ref, dinv_ref, b_ref, w_ref, sqc_ref, o_ref, os_ref):
        dinv_b = dinv_ref[...]
        sqc_s = sqc_ref[0, 0]
        t = (sqc_s * (p_ref[0] + p_ref[1]) + g_ref[...]) * dinv_b + b_ref[...]
        t = jnp.maximum(t, 0.0)
        g = jnp.dot(t, w_ref[...], preferred_element_type=jnp.float32) * dinv_b
        o_ref[...] = g
        os_ref[...] = g * sqc_s

    return pl.pallas_call(
        body,
        grid=(N // B,),
        in_specs=[
            pl.BlockSpec((NC, B, D), lambda i: (0, i, 0)),
            pl.BlockSpec((B, D), lambda i: (i, 0)),
            pl.BlockSpec((B, 1), lambda i: (i, 0)),
            pl.BlockSpec((1, D), lambda i: (0, 0)),
            pl.BlockSpec((D, D), lambda i: (0, 0)),
            pl.BlockSpec((1, 1), lambda i: (0, 0)),
        ],
        out_specs=[
            pl.BlockSpec((B, D), lambda i: (i, 0)),
            pl.BlockSpec((B, D), lambda i: (i, 0)),
        ],
        out_shape=[
            jax.ShapeDtypeStruct((N, D), jnp.float32),
            jax.ShapeDtypeStruct((N, D), jnp.float32),
        ],
    )(p, g, dinv, b.reshape(1, D), Wn, sqc)


def _final_tc(p, g, dinv, b, sqc, N, D):
    B = 2000

    def body(p_ref, g_ref, dinv_ref, b_ref, sqc_ref, o_ref):
        o_ref[...] = (
            sqc_ref[0, 0] * (p_ref[0] + p_ref[1]) + g_ref[...]
        ) * dinv_ref[...] + b_ref[...]

    return pl.pallas_call(
        body,
        grid=(N // B,),
        in_specs=[
            pl.BlockSpec((NC, B, D), lambda i: (0, i, 0)),
            pl.BlockSpec((B, D), lambda i: (i, 0)),
            pl.BlockSpec((B, 1), lambda i: (i, 0)),
            pl.BlockSpec((1, D), lambda i: (0, 0)),
            pl.BlockSpec((1, 1), lambda i: (0, 0)),
        ],
        out_specs=pl.BlockSpec((B, D), lambda i: (i, 0)),
        out_shape=jax.ShapeDtypeStruct((N, D), jnp.float32),
    )(p, g, dinv, b.reshape(1, D), sqc)


def kernel(x, edge_index, edge_weights, W1, b1, W2, b2, W3, b3):
    N, D = x.shape
    E = edge_weights.shape[0]
    EPW = E // NW
    CHUNKS = EPW // K

    src = edge_index[0]
    dst = edge_index[1]

    NB = 5
    CB = CHUNKS // NB
    w, flag, sqc = _softplus_tc(edge_weights, E)
    w_flat = w.reshape(E)
    flag_flat = flag.reshape(128)
    src4 = src.reshape(NW, NB, CB, K)
    dst4 = dst.reshape(NW, NB, CB, K)
    w4 = w_flat.reshape(NW, NB, CB, K)

    dp = _deg_sc(dst, w_flat, N, E)
    g1, gs1, dinv = _prep_tc(dp, x, W1, sqc, N, D)
    p1 = _spmm_sc(gs1, src4, dst4, w4, flag_flat, N, D)
    g2, gs2 = _mid_tc(p1, g1, dinv, b1, W2, sqc, N, D)
    p2 = _spmm_sc(gs2, src4, dst4, w4, flag_flat, N, D)
    g3, gs3 = _mid_tc(p2, g2, dinv, b2, W3, sqc, N, D)
    p3 = _spmm_sc(gs3, src4, dst4, w4, flag_flat, N, D)
    out = _final_tc(p3, g3, dinv, b3, sqc, N, D)
    return out
